# baseline (speedup 1.0000x reference)
import functools

import jax
import jax.numpy as jnp
from jax import lax
from jax.experimental import pallas as pl
from jax.experimental.pallas import tpu as pltpu
from jax.experimental.pallas import tpu_sc as plsc

N = 10000
E = 160000
D = 256
H = D // 2
MAX_NB = 16

NC = 2
NS = 16
NW = NC * NS

N_PAD = 10240
APW = N_PAD // NW
CA = 8
EPW = 5120
EPW_LAST = E - 31 * EPW
CE = 64
RU = 8

_MESH = plsc.VectorSubcoreMesh(core_axis_name="c", subcore_axis_name="s",
                               num_cores=NC, num_subcores=NS)


def _wid():
    return lax.axis_index("s") * NC + lax.axis_index("c")


def _copy_idx(src_ref, src_off, dst_ref, n):
    for k in range(n // 16):
        dst_ref[pl.ds(k * 16, 16)] = src_ref[pl.ds(src_off + k * 16, 16)]


def _pipelined(nchunks, issue, process):
    issue(0, 0)
    if nchunks > 1:
        issue(1, 1)
    npairs = (nchunks + 1) // 2

    def body(j, carry):
        i0 = 2 * j
        process(i0, 0)

        @pl.when(i0 + 2 < nchunks)
        def _():
            issue(i0 + 2, 0)

        @pl.when(i0 + 1 < nchunks)
        def _():
            process(i0 + 1, 1)

        @pl.when(i0 + 3 < nchunks)
        def _():
            issue(i0 + 3, 1)

        return carry

    lax.fori_loop(0, npairs, body, 0, unroll=False)


@functools.partial(
    pl.kernel,
    out_type=jax.ShapeDtypeStruct((N_PAD, D), jnp.float32),
    mesh=_MESH,
    scratch_types=[
        pltpu.VMEM((APW * MAX_NB,), jnp.int32),
        pltpu.VMEM((CA * MAX_NB,), jnp.int32),
        pltpu.VMEM((CA * MAX_NB,), jnp.int32),
        pltpu.VMEM((2, CA * MAX_NB, D), jnp.float32),
        pltpu.VMEM((2, CA, D), jnp.float32),
        pltpu.SemaphoreType.DMA,
        pltpu.SemaphoreType.DMA,
        pltpu.SemaphoreType.DMA,
        pltpu.SemaphoreType.DMA,
    ],
)
def _aggregate(a2b_hbm, edge_attr_hbm, amsg_hbm, idx_v, cidx0, cidx1,
               rows_v, acc_v, sem0, sem1, osem0, osem1):
    wid = _wid()
    abase = wid * APW
    pltpu.sync_copy(a2b_hbm.at[pl.ds(abase * MAX_NB, APW * MAX_NB)], idx_v)
    cidxs = (cidx0, cidx1)
    sems = (sem0, sem1)
    osems = (osem0, osem1)
    nchunks = APW // CA

    def issue(i, slot):
        _copy_idx(idx_v, i * CA * MAX_NB, cidxs[slot], CA * MAX_NB)
        pltpu.async_copy(edge_attr_hbm.at[cidxs[slot]], rows_v.at[slot],
                         sems[slot])

    def process(i, slot):
        pltpu.make_async_copy(
            edge_attr_hbm.at[pl.ds(0, CA * MAX_NB)], rows_v.at[slot],
            sems[slot]).wait()

        @pl.when(i >= 2)
        def _():
            pltpu.make_async_copy(
                acc_v.at[slot], amsg_hbm.at[pl.ds(0, CA)], osems[slot]).wait()

        def atom(a, c2):
            r0 = a * MAX_NB
            for col in range(D // 16):
                s = pl.ds(col * 16, 16)
                acc = rows_v[slot, r0, s]
                for nb in range(1, MAX_NB):
                    acc = acc + rows_v[slot, r0 + nb, s]
                acc_v[slot, a, s] = acc
            return c2

        lax.fori_loop(0, CA, atom, 0, unroll=False)
        pltpu.async_copy(acc_v.at[slot],
                         amsg_hbm.at[pl.ds(abase + i * CA, CA)], osems[slot])

    _pipelined(nchunks, issue, process)
    pltpu.make_async_copy(acc_v.at[0], amsg_hbm.at[pl.ds(0, CA)], osem0).wait()
    pltpu.make_async_copy(acc_v.at[1], amsg_hbm.at[pl.ds(0, CA)], osem1).wait()


def _mm_pack_body(x_ref, w_ref, b_ref, o_ref):
    x = x_ref[...].astype(jnp.bfloat16)
    o = lax.dot_general(x, w_ref[...], (((1,), (0,)), ((), ())),
                        preferred_element_type=jnp.float32) + b_ref[...]
    ulo = lax.bitcast_convert_type(o[:, :H], jnp.uint32)
    uhi = lax.bitcast_convert_type(o[:, H:], jnp.uint32)
    half = jnp.uint32(0x8000)
    packed = ((ulo + half) >> 16) | ((uhi + half) & jnp.uint32(0xFFFF0000))
    o_ref[...] = lax.bitcast_convert_type(packed, jnp.int32)


def _mm_pack(xin, wt_bf16, bias, rows, block):
    return pl.pallas_call(
        _mm_pack_body,
        grid=(rows // block,),
        in_specs=[
            pl.BlockSpec((block, D), lambda i: (i, 0)),
            pl.BlockSpec((D, D), lambda i: (0, 0)),
            pl.BlockSpec((1, D), lambda i: (0, 0)),
        ],
        out_specs=pl.BlockSpec((block, H), lambda i: (i, 0)),
        out_shape=jax.ShapeDtypeStruct((rows, H), jnp.int32),
    )(xin, wt_bf16, bias)


@functools.partial(
    pl.kernel,
    out_type=jax.ShapeDtypeStruct((E, D), jnp.float32),
    mesh=_MESH,
    scratch_types=[
        pltpu.VMEM((EPW,), jnp.int32),
        pltpu.VMEM((EPW,), jnp.int32),
        pltpu.VMEM((CE,), jnp.int32),
        pltpu.VMEM((CE,), jnp.int32),
        pltpu.VMEM((CE,), jnp.int32),
        pltpu.VMEM((CE,), jnp.int32),
        pltpu.VMEM((2, CE, H), jnp.int32),
        pltpu.VMEM((2, CE, H), jnp.int32),
        pltpu.VMEM((2, CE, D), jnp.float32),
        pltpu.SemaphoreType.DMA,
        pltpu.SemaphoreType.DMA,
        pltpu.SemaphoreType.DMA,
        pltpu.SemaphoreType.DMA,
    ],
)
def _combine(b2a_hbm, b2revb_hbm, ta_hbm, te_hbm, out_hbm,
             idxa_v, idxr_v, ca0, ca1, cr0, cr1, ra_v, rr_v, ro_v,
             sem0, sem1, osem0, osem1):
    wid = _wid()
    ebase = wid * EPW
    cas = (ca0, ca1)
    crs = (cr0, cr1)
    sems = (sem0, sem1)
    osems = (osem0, osem1)
    hi_mask = jnp.int32(-65536)

    def run(epw_here):
        pltpu.sync_copy(b2a_hbm.at[pl.ds(ebase, epw_here)],
                        idxa_v.at[pl.ds(0, epw_here)])
        pltpu.sync_copy(b2revb_hbm.at[pl.ds(ebase, epw_here)],
                        idxr_v.at[pl.ds(0, epw_here)])
        nchunks = epw_here // CE

        def issue(i, slot):
            e0 = i * CE
            _copy_idx(idxa_v, e0, cas[slot], CE)
            _copy_idx(idxr_v, e0, crs[slot], CE)
            pltpu.async_copy(ta_hbm.at[cas[slot]], ra_v.at[slot], sems[slot])
            pltpu.async_copy(te_hbm.at[crs[slot]], rr_v.at[slot], sems[slot])

        def process(i, slot):
            pltpu.make_async_copy(
                ta_hbm.at[pl.ds(0, CE)], ra_v.at[slot], sems[slot]).wait()
            pltpu.make_async_copy(
                te_hbm.at[pl.ds(0, CE)], rr_v.at[slot], sems[slot]).wait()

            @pl.when(i >= 2)
            def _():
                pltpu.make_async_copy(
                    ro_v.at[slot], out_hbm.at[pl.ds(0, CE)],
                    osems[slot]).wait()

            def rowblk(j, c2):
                r0 = j * RU
                for k in range(RU):
                    r = r0 + k
                    for col in range(H // 16):
                        s = pl.ds(col * 16, 16)
                        ua = ra_v[slot, r, s]
                        ue = rr_v[slot, r, s]
                        lo = lax.bitcast_convert_type(ua << 16, jnp.float32) \
                            - lax.bitcast_convert_type(ue << 16, jnp.float32)
                        hi = lax.bitcast_convert_type(ua & hi_mask,
                                                      jnp.float32) \
                            - lax.bitcast_convert_type(ue & hi_mask,
                                                       jnp.float32)
                        ro_v[slot, r, pl.ds(col * 16, 16)] = lo
                        ro_v[slot, r, pl.ds(H + col * 16, 16)] = hi
                return c2

            lax.fori_loop(0, CE // RU, rowblk, 0, unroll=False)
            pltpu.async_copy(ro_v.at[slot],
                             out_hbm.at[pl.ds(ebase + i * CE, CE)],
                             osems[slot])

        _pipelined(nchunks, issue, process)
        pltpu.make_async_copy(ro_v.at[0], out_hbm.at[pl.ds(0, CE)],
                              osem0).wait()
        pltpu.make_async_copy(ro_v.at[1], out_hbm.at[pl.ds(0, CE)],
                              osem1).wait()

    @pl.when(wid < NW - 1)
    def _():
        run(EPW)

    @pl.when(wid == NW - 1)
    def _():
        run(EPW_LAST)


def kernel(x, edge_index, edge_attr, a2b, b2a, b2revb, W, b):
    del x, edge_index
    wt = W.T.astype(jnp.bfloat16)
    zero_bias = jnp.zeros((1, D), jnp.float32)
    te = _mm_pack(edge_attr, wt, zero_bias, E, 2000)
    a2b_flat = jnp.pad(a2b.reshape(-1), (0, (N_PAD - N) * MAX_NB))
    amsg = _aggregate(a2b_flat, edge_attr)
    ta = _mm_pack(amsg, wt, b.reshape(1, D), N_PAD, 1280)
    return _combine(b2a, b2revb, ta, te)


# speedup vs baseline: 1.1032x; 1.1032x over previous
import functools

import jax
import jax.numpy as jnp
from jax import lax
from jax.experimental import pallas as pl
from jax.experimental.pallas import tpu as pltpu
from jax.experimental.pallas import tpu_sc as plsc

N = 10000
E = 160000
D = 256
MAX_NB = 16

NC = 2
NS = 16
NW = NC * NS

N_PAD = 10240
APW = N_PAD // NW
CA = 8
EPW = E // NW
CE = 40

_MESH = plsc.VectorSubcoreMesh(core_axis_name="c", subcore_axis_name="s",
                               num_cores=NC, num_subcores=NS)


def _wid():
    return lax.axis_index("s") * NC + lax.axis_index("c")


def _pipelined(nchunks, issue, process):
    issue(0, 0)
    if nchunks > 1:
        issue(1, 1)
    npairs = (nchunks + 1) // 2

    def body(j, carry):
        i0 = 2 * j
        process(i0, 0)

        @pl.when(i0 + 2 < nchunks)
        def _():
            issue(i0 + 2, 0)

        @pl.when(i0 + 1 < nchunks)
        def _():
            process(i0 + 1, 1)

        @pl.when(i0 + 3 < nchunks)
        def _():
            issue(i0 + 3, 1)

        return carry

    lax.fori_loop(0, npairs, body, 0, unroll=False)


@functools.partial(
    pl.kernel,
    out_type=jax.ShapeDtypeStruct((N_PAD, D), jnp.float32),
    mesh=_MESH,
    scratch_types=[
        pltpu.VMEM((APW * MAX_NB,), jnp.int32),
        pltpu.VMEM((2, CA * MAX_NB, D), jnp.float32),
        pltpu.VMEM((2, CA, D), jnp.float32),
        pltpu.SemaphoreType.DMA,
        pltpu.SemaphoreType.DMA,
        pltpu.SemaphoreType.DMA,
        pltpu.SemaphoreType.DMA,
    ],
)
def _aggregate(a2b_hbm, edge_attr_hbm, amsg_hbm, idx_v, rows_v, acc_v,
               sem0, sem1, osem0, osem1):
    wid = _wid()
    abase = wid * APW
    pltpu.sync_copy(a2b_hbm.at[pl.ds(abase * MAX_NB, APW * MAX_NB)], idx_v)
    sems = (sem0, sem1)
    osems = (osem0, osem1)
    nchunks = APW // CA

    def issue(i, slot):
        pltpu.async_copy(
            edge_attr_hbm.at[idx_v.at[pl.ds(i * CA * MAX_NB, CA * MAX_NB)]],
            rows_v.at[slot], sems[slot])

    def process(i, slot):
        pltpu.make_async_copy(
            edge_attr_hbm.at[pl.ds(0, CA * MAX_NB)], rows_v.at[slot],
            sems[slot]).wait()

        @pl.when(i >= 2)
        def _():
            pltpu.make_async_copy(
                acc_v.at[slot], amsg_hbm.at[pl.ds(0, CA)], osems[slot]).wait()

        def atom(a, c2):
            r0 = a * MAX_NB
            for col in range(D // 16):
                s = pl.ds(col * 16, 16)
                acc = rows_v[slot, r0, s]
                for nb in range(1, MAX_NB):
                    acc = acc + rows_v[slot, r0 + nb, s]
                acc_v[slot, a, s] = acc
            return c2

        lax.fori_loop(0, CA, atom, 0, unroll=False)
        pltpu.async_copy(acc_v.at[slot],
                         amsg_hbm.at[pl.ds(abase + i * CA, CA)], osems[slot])

    _pipelined(nchunks, issue, process)
    pltpu.make_async_copy(acc_v.at[0], amsg_hbm.at[pl.ds(0, CA)], osem0).wait()
    pltpu.make_async_copy(acc_v.at[1], amsg_hbm.at[pl.ds(0, CA)], osem1).wait()


def _mm_body(x_ref, w_ref, b_ref, o_ref):
    x = x_ref[...].astype(jnp.bfloat16)
    o_ref[...] = lax.dot_general(
        x, w_ref[...], (((1,), (0,)), ((), ())),
        preferred_element_type=jnp.float32) + b_ref[...]


def _mm(xin, wt_bf16, bias, rows, block):
    return pl.pallas_call(
        _mm_body,
        grid=(rows // block,),
        in_specs=[
            pl.BlockSpec((block, D), lambda i: (i, 0)),
            pl.BlockSpec((D, D), lambda i: (0, 0)),
            pl.BlockSpec((1, D), lambda i: (0, 0)),
        ],
        out_specs=pl.BlockSpec((block, D), lambda i: (i, 0)),
        out_shape=jax.ShapeDtypeStruct((rows, D), jnp.float32),
    )(xin, wt_bf16, bias)


@functools.partial(
    pl.kernel,
    out_type=jax.ShapeDtypeStruct((E, D), jnp.float32),
    mesh=_MESH,
    scratch_types=[
        pltpu.VMEM((EPW,), jnp.int32),
        pltpu.VMEM((EPW,), jnp.int32),
        pltpu.VMEM((2, CE, D), jnp.float32),
        pltpu.VMEM((2, CE, D), jnp.float32),
        pltpu.VMEM((2, CE, D), jnp.float32),
        pltpu.SemaphoreType.DMA,
        pltpu.SemaphoreType.DMA,
        pltpu.SemaphoreType.DMA,
        pltpu.SemaphoreType.DMA,
    ],
)
def _combine(b2a_hbm, b2revb_hbm, ta_hbm, te_hbm, out_hbm,
             idxa_v, idxr_v, ra_v, rr_v, ro_v, sem0, sem1, osem0, osem1):
    wid = _wid()
    ebase = wid * EPW
    pltpu.sync_copy(b2a_hbm.at[pl.ds(ebase, EPW)], idxa_v)
    pltpu.sync_copy(b2revb_hbm.at[pl.ds(ebase, EPW)], idxr_v)
    sems = (sem0, sem1)
    osems = (osem0, osem1)
    nchunks = EPW // CE

    def issue(i, slot):
        e0 = i * CE
        pltpu.async_copy(ta_hbm.at[idxa_v.at[pl.ds(e0, CE)]],
                         ra_v.at[slot], sems[slot])
        pltpu.async_copy(te_hbm.at[idxr_v.at[pl.ds(e0, CE)]],
                         rr_v.at[slot], sems[slot])

    def process(i, slot):
        pltpu.make_async_copy(
            ta_hbm.at[pl.ds(0, CE)], ra_v.at[slot], sems[slot]).wait()
        pltpu.make_async_copy(
            te_hbm.at[pl.ds(0, CE)], rr_v.at[slot], sems[slot]).wait()

        @pl.when(i >= 2)
        def _():
            pltpu.make_async_copy(
                ro_v.at[slot], out_hbm.at[pl.ds(0, CE)], osems[slot]).wait()

        def row(r, c2):
            for col in range(D // 16):
                s = pl.ds(col * 16, 16)
                ro_v[slot, r, s] = ra_v[slot, r, s] - rr_v[slot, r, s]
            return c2

        lax.fori_loop(0, CE, row, 0, unroll=False)
        pltpu.async_copy(ro_v.at[slot],
                         out_hbm.at[pl.ds(ebase + i * CE, CE)], osems[slot])

    _pipelined(nchunks, issue, process)
    pltpu.make_async_copy(ro_v.at[0], out_hbm.at[pl.ds(0, CE)], osem0).wait()
    pltpu.make_async_copy(ro_v.at[1], out_hbm.at[pl.ds(0, CE)], osem1).wait()


def kernel(x, edge_index, edge_attr, a2b, b2a, b2revb, W, b):
    del x, edge_index
    wt = W.T.astype(jnp.bfloat16)
    zero_bias = jnp.zeros((1, D), jnp.float32)
    te = _mm(edge_attr, wt, zero_bias, E, 2000)
    a2b_flat = jnp.pad(a2b.reshape(-1), (0, (N_PAD - N) * MAX_NB))
    amsg = _aggregate(a2b_flat, edge_attr)
    ta = _mm(amsg, wt, b.reshape(1, D), N_PAD, 1280)
    return _combine(b2a, b2revb, ta, te)


# speedup vs baseline: 1.1042x; 1.0009x over previous
import functools

import jax
import jax.numpy as jnp
from jax import lax
from jax.experimental import pallas as pl
from jax.experimental.pallas import tpu as pltpu
from jax.experimental.pallas import tpu_sc as plsc

N = 10000
E = 160000
D = 256
MAX_NB = 16

NC = 2
NS = 16
NW = NC * NS

N_PAD = 10240
APW = N_PAD // NW
CA = 8
EPW = E // NW
CE = 40

_MESH = plsc.VectorSubcoreMesh(core_axis_name="c", subcore_axis_name="s",
                               num_cores=NC, num_subcores=NS)


def _wid():
    return lax.axis_index("s") * NC + lax.axis_index("c")


def _pipelined(nchunks, issue, process, nbuf=2):
    for slot in range(nbuf):
        if slot < nchunks:
            issue(slot, slot)
    ngroups = (nchunks + nbuf - 1) // nbuf

    def body(j, carry):
        i0 = nbuf * j
        for k in range(nbuf):
            i = i0 + k

            def _go(i=i, k=k):
                process(i, k)

                @pl.when(i + nbuf < nchunks)
                def _():
                    issue(i + nbuf, k)

            if k == 0:
                _go()
            else:
                pl.when(i < nchunks)(_go)

        return carry

    lax.fori_loop(0, ngroups, body, 0, unroll=False)


@functools.partial(
    pl.kernel,
    out_type=jax.ShapeDtypeStruct((N_PAD, D), jnp.float32),
    mesh=_MESH,
    scratch_types=[
        pltpu.VMEM((APW * MAX_NB,), jnp.int32),
        pltpu.VMEM((3, CA * MAX_NB, D), jnp.float32),
        pltpu.VMEM((3, CA, D), jnp.float32),
        pltpu.SemaphoreType.DMA,
        pltpu.SemaphoreType.DMA,
        pltpu.SemaphoreType.DMA,
        pltpu.SemaphoreType.DMA,
        pltpu.SemaphoreType.DMA,
        pltpu.SemaphoreType.DMA,
    ],
)
def _aggregate(a2b_hbm, edge_attr_hbm, amsg_hbm, idx_v, rows_v, acc_v,
               sem0, sem1, sem2, osem0, osem1, osem2):
    wid = _wid()
    abase = wid * APW
    pltpu.sync_copy(a2b_hbm.at[pl.ds(abase * MAX_NB, APW * MAX_NB)], idx_v)
    sems = (sem0, sem1, sem2)
    osems = (osem0, osem1, osem2)
    nchunks = APW // CA

    def issue(i, slot):
        pltpu.async_copy(
            edge_attr_hbm.at[idx_v.at[pl.ds(i * CA * MAX_NB, CA * MAX_NB)]],
            rows_v.at[slot], sems[slot])

    def process(i, slot):
        pltpu.make_async_copy(
            edge_attr_hbm.at[pl.ds(0, CA * MAX_NB)], rows_v.at[slot],
            sems[slot]).wait()

        @pl.when(i >= 3)
        def _():
            pltpu.make_async_copy(
                acc_v.at[slot], amsg_hbm.at[pl.ds(0, CA)], osems[slot]).wait()

        def atom(a, c2):
            r0 = a * MAX_NB
            for col in range(D // 16):
                s = pl.ds(col * 16, 16)
                acc = rows_v[slot, r0, s]
                for nb in range(1, MAX_NB):
                    acc = acc + rows_v[slot, r0 + nb, s]
                acc_v[slot, a, s] = acc
            return c2

        lax.fori_loop(0, CA, atom, 0, unroll=False)
        pltpu.async_copy(acc_v.at[slot],
                         amsg_hbm.at[pl.ds(abase + i * CA, CA)], osems[slot])

    _pipelined(nchunks, issue, process, nbuf=3)
    pltpu.make_async_copy(acc_v.at[0], amsg_hbm.at[pl.ds(0, CA)], osem0).wait()
    pltpu.make_async_copy(acc_v.at[1], amsg_hbm.at[pl.ds(0, CA)], osem1).wait()
    pltpu.make_async_copy(acc_v.at[2], amsg_hbm.at[pl.ds(0, CA)], osem2).wait()


def _mm_body(x_ref, w_ref, b_ref, o_ref):
    x = x_ref[...].astype(jnp.bfloat16)
    o_ref[...] = lax.dot_general(
        x, w_ref[...], (((1,), (0,)), ((), ())),
        preferred_element_type=jnp.float32) + b_ref[...]


def _mm(xin, wt_bf16, bias, rows, block):
    return pl.pallas_call(
        _mm_body,
        grid=(rows // block,),
        in_specs=[
            pl.BlockSpec((block, D), lambda i: (i, 0)),
            pl.BlockSpec((D, D), lambda i: (0, 0)),
            pl.BlockSpec((1, D), lambda i: (0, 0)),
        ],
        out_specs=pl.BlockSpec((block, D), lambda i: (i, 0)),
        out_shape=jax.ShapeDtypeStruct((rows, D), jnp.float32),
    )(xin, wt_bf16, bias)


@functools.partial(
    pl.kernel,
    out_type=jax.ShapeDtypeStruct((E, D), jnp.float32),
    mesh=_MESH,
    scratch_types=[
        pltpu.VMEM((EPW,), jnp.int32),
        pltpu.VMEM((EPW,), jnp.int32),
        pltpu.VMEM((2, CE, D), jnp.float32),
        pltpu.VMEM((2, CE, D), jnp.float32),
        pltpu.VMEM((2, CE, D), jnp.float32),
        pltpu.SemaphoreType.DMA,
        pltpu.SemaphoreType.DMA,
        pltpu.SemaphoreType.DMA,
        pltpu.SemaphoreType.DMA,
    ],
)
def _combine(b2a_hbm, b2revb_hbm, ta_hbm, te_hbm, out_hbm,
             idxa_v, idxr_v, ra_v, rr_v, ro_v, sem0, sem1, osem0, osem1):
    wid = _wid()
    ebase = wid * EPW
    pltpu.sync_copy(b2a_hbm.at[pl.ds(ebase, EPW)], idxa_v)
    pltpu.sync_copy(b2revb_hbm.at[pl.ds(ebase, EPW)], idxr_v)
    sems = (sem0, sem1)
    osems = (osem0, osem1)
    nchunks = EPW // CE

    def issue(i, slot):
        e0 = i * CE
        pltpu.async_copy(ta_hbm.at[idxa_v.at[pl.ds(e0, CE)]],
                         ra_v.at[slot], sems[slot])
        pltpu.async_copy(te_hbm.at[idxr_v.at[pl.ds(e0, CE)]],
                         rr_v.at[slot], sems[slot])

    def process(i, slot):
        pltpu.make_async_copy(
            ta_hbm.at[pl.ds(0, CE)], ra_v.at[slot], sems[slot]).wait()
        pltpu.make_async_copy(
            te_hbm.at[pl.ds(0, CE)], rr_v.at[slot], sems[slot]).wait()

        @pl.when(i >= 2)
        def _():
            pltpu.make_async_copy(
                ro_v.at[slot], out_hbm.at[pl.ds(0, CE)], osems[slot]).wait()

        def row(r, c2):
            for col in range(D // 16):
                s = pl.ds(col * 16, 16)
                ro_v[slot, r, s] = ra_v[slot, r, s] - rr_v[slot, r, s]
            return c2

        lax.fori_loop(0, CE, row, 0, unroll=False)
        pltpu.async_copy(ro_v.at[slot],
                         out_hbm.at[pl.ds(ebase + i * CE, CE)], osems[slot])

    _pipelined(nchunks, issue, process)
    pltpu.make_async_copy(ro_v.at[0], out_hbm.at[pl.ds(0, CE)], osem0).wait()
    pltpu.make_async_copy(ro_v.at[1], out_hbm.at[pl.ds(0, CE)], osem1).wait()


def kernel(x, edge_index, edge_attr, a2b, b2a, b2revb, W, b):
    del x, edge_index
    wt = W.T.astype(jnp.bfloat16)
    zero_bias = jnp.zeros((1, D), jnp.float32)
    te = _mm(edge_attr, wt, zero_bias, E, 2000)
    a2b_flat = jnp.pad(a2b.reshape(-1), (0, (N_PAD - N) * MAX_NB))
    amsg = _aggregate(a2b_flat, edge_attr)
    ta = _mm(amsg, wt, b.reshape(1, D), N_PAD, 1280)
    return _combine(b2a, b2revb, ta, te)
